# Initial kernel scaffold; baseline (speedup 1.0000x reference)
import jax
import jax.numpy as jnp
from jax.experimental import pallas as pl


def kernel(x, edge_index, W, b):
    raise NotImplementedError("write your pallas kernel here")


# speedup vs baseline: 15.5613x; 15.5613x over previous
import functools

import jax
import jax.numpy as jnp
from jax import lax
from jax.experimental import pallas as pl
from jax.experimental.pallas import tpu as pltpu
from jax.experimental.pallas import tpu_sc as plsc

_NC = 2
_NS = 16
_NW = _NC * _NS
_B = 80
_DEGW = 128


def _sc_mesh():
    return plsc.VectorSubcoreMesh(core_axis_name="c", subcore_axis_name="s")


def _pad_rows(n):
    step = 8 * _NS
    return ((n + step - 1) // step) * step


def _make_deg_kernel(n_pad, e):
    rpt = n_pad // _NS
    ept = e // _NW
    nchunk = ept // _B

    @functools.partial(
        pl.kernel,
        out_type=jax.ShapeDtypeStruct((_NC, n_pad, _DEGW), jnp.float32),
        mesh=_sc_mesh(),
        scratch_types=[
            pltpu.VMEM((_B,), jnp.int32),
            pltpu.VMEM((_B, _DEGW), jnp.float32),
            pltpu.VMEM_SHARED((n_pad, _DEGW), jnp.float32),
        ],
    )
    def deg_kernel(dst_hbm, zeros_hbm, ones_hbm, out_hbm, idx_v, ones_v, deg_sp):
        c = lax.axis_index("c")
        s = lax.axis_index("s")
        wid = c * _NS + s
        r0 = s * rpt
        pltpu.sync_copy(zeros_hbm, deg_sp.at[pl.ds(r0, rpt)])
        pltpu.sync_copy(ones_hbm, ones_v)
        plsc.subcore_barrier()
        e0 = wid * ept

        def body(i, carry):
            base = e0 + i * _B
            pltpu.sync_copy(dst_hbm.at[pl.ds(base, _B)], idx_v)
            pltpu.sync_copy(ones_v, deg_sp.at[idx_v], add=True)
            return carry

        lax.fori_loop(0, nchunk, body, 0)
        plsc.subcore_barrier()
        pltpu.sync_copy(deg_sp.at[pl.ds(r0, rpt)], out_hbm.at[c].at[pl.ds(r0, rpt)])

    return deg_kernel


def _make_scatter_kernel(n_pad, e, d):
    rpt = n_pad // _NS
    ept = e // _NW
    nchunk = ept // _B

    @functools.partial(
        pl.kernel,
        out_type=jax.ShapeDtypeStruct((_NC, n_pad, d), jnp.float32),
        mesh=_sc_mesh(),
        scratch_types=[
            pltpu.VMEM((_B,), jnp.int32),
            pltpu.VMEM((_B,), jnp.int32),
            pltpu.VMEM((_B, d), jnp.float32),
            pltpu.VMEM_SHARED((n_pad, d), jnp.float32),
            pltpu.SemaphoreType.DMA,
        ],
    )
    def scatter_kernel(src_hbm, dst_hbm, g_hbm, zeros_hbm, out_hbm,
                       sidx_v, didx_v, rows_v, acc_sp, sem):
        c = lax.axis_index("c")
        s = lax.axis_index("s")
        wid = c * _NS + s
        r0 = s * rpt
        pltpu.sync_copy(zeros_hbm, acc_sp.at[pl.ds(r0, rpt)])
        plsc.subcore_barrier()
        e0 = wid * ept

        def body(i, carry):
            base = e0 + i * _B
            pltpu.sync_copy(src_hbm.at[pl.ds(base, _B)], sidx_v)
            pltpu.sync_copy(dst_hbm.at[pl.ds(base, _B)], didx_v)
            pltpu.async_copy(g_hbm.at[sidx_v], rows_v, sem).wait()
            pltpu.sync_copy(rows_v, acc_sp.at[didx_v], add=True)
            return carry

        lax.fori_loop(0, nchunk, body, 0)
        plsc.subcore_barrier()
        pltpu.sync_copy(acc_sp.at[pl.ds(r0, rpt)], out_hbm.at[c].at[pl.ds(r0, rpt)])

    return scatter_kernel


def _tc_scale_body(x_ref, w_ref, dp_ref, g_ref):
    h = jnp.dot(x_ref[...], w_ref[...], preferred_element_type=jnp.float32)
    deg = dp_ref[0, :, 0:1] + dp_ref[1, :, 0:1] + 1.0
    g_ref[...] = h * lax.rsqrt(deg)


def _tc_final_body(s_ref, g_ref, dp_ref, b_ref, o_ref):
    deg = dp_ref[0, :, 0:1] + dp_ref[1, :, 0:1] + 1.0
    dinv = lax.rsqrt(deg)
    agg = s_ref[0] + s_ref[1] + g_ref[...]
    o_ref[...] = jnp.maximum(dinv * agg + b_ref[...], 0.0)


def kernel(x, edge_index, W, b):
    n, d = x.shape
    e = edge_index.shape[1]
    src = edge_index[0]
    dst = edge_index[1]
    n_pad = _pad_rows(n)
    rpt = n_pad // _NS

    zeros16 = jnp.zeros((rpt, _DEGW), jnp.float32)
    ones16 = jnp.ones((_B, _DEGW), jnp.float32)
    zeros_d = jnp.zeros((rpt, d), jnp.float32)

    deg_partials = _make_deg_kernel(n_pad, e)(dst, zeros16, ones16)
    deg_partials = deg_partials[:, :n, :]

    blk = 2000
    grid = n // blk
    g = pl.pallas_call(
        _tc_scale_body,
        out_shape=jax.ShapeDtypeStruct((n, d), jnp.float32),
        grid=(grid,),
        in_specs=[
            pl.BlockSpec((blk, d), lambda i: (i, 0)),
            pl.BlockSpec((d, d), lambda i: (0, 0)),
            pl.BlockSpec((_NC, blk, _DEGW), lambda i: (0, i, 0)),
        ],
        out_specs=pl.BlockSpec((blk, d), lambda i: (i, 0)),
    )(x, W, deg_partials)

    s_partials = _make_scatter_kernel(n_pad, e, d)(src, dst, g, zeros_d)
    s_partials = s_partials[:, :n, :]

    out = pl.pallas_call(
        _tc_final_body,
        out_shape=jax.ShapeDtypeStruct((n, d), jnp.float32),
        grid=(grid,),
        in_specs=[
            pl.BlockSpec((_NC, blk, d), lambda i: (0, i, 0)),
            pl.BlockSpec((blk, d), lambda i: (i, 0)),
            pl.BlockSpec((_NC, blk, _DEGW), lambda i: (0, i, 0)),
            pl.BlockSpec((1, d), lambda i: (0, 0)),
        ],
        out_specs=pl.BlockSpec((blk, d), lambda i: (i, 0)),
    )(s_partials, g, deg_partials, b.reshape(1, d))

    return out


# speedup vs baseline: 26.9423x; 1.7314x over previous
import functools

import jax
import jax.numpy as jnp
from jax import lax
from jax.experimental import pallas as pl
from jax.experimental.pallas import tpu as pltpu
from jax.experimental.pallas import tpu_sc as plsc

_NC = 2
_NS = 16
_NW = _NC * _NS
_B = 80
_DEGW = 128


def _sc_mesh():
    return plsc.VectorSubcoreMesh(core_axis_name="c", subcore_axis_name="s")


def _pad_rows(n):
    step = 8 * _NS
    return ((n + step - 1) // step) * step


_K = 5


def _make_deg_kernel(n_pad, e):
    rpt = n_pad // _NS
    ept = e // _NW
    nsuper = ept // (_B * _K)

    @functools.partial(
        pl.kernel,
        out_type=jax.ShapeDtypeStruct((_NC, n_pad, _DEGW), jnp.float32),
        mesh=_sc_mesh(),
        scratch_types=(
            [pltpu.VMEM((_B,), jnp.int32) for _ in range(_K)]
            + [
                pltpu.VMEM((_B, _DEGW), jnp.float32),
                pltpu.VMEM_SHARED((n_pad, _DEGW), jnp.float32),
                pltpu.SemaphoreType.DMA,
                pltpu.SemaphoreType.DMA,
            ]
        ),
    )
    def deg_kernel(dst_hbm, zeros_hbm, ones_hbm, out_hbm, *scr):
        didx = scr[:_K]
        ones_v, deg_sp, isem, ssem = scr[_K:]
        c = lax.axis_index("c")
        s = lax.axis_index("s")
        wid = c * _NS + s
        r0 = s * rpt
        pltpu.sync_copy(zeros_hbm, deg_sp.at[pl.ds(r0, rpt)])
        pltpu.sync_copy(ones_hbm, ones_v)
        plsc.subcore_barrier()
        e0 = wid * ept

        def body(m, carry):
            base = e0 + m * (_B * _K)
            ic = [
                pltpu.async_copy(dst_hbm.at[pl.ds(base + i * _B, _B)], didx[i], isem)
                for i in range(_K)
            ]
            for d in ic:
                d.wait()
            sc = [
                pltpu.async_copy(ones_v, deg_sp.at[didx[i]], ssem, add=True)
                for i in range(_K)
            ]
            for d in sc:
                d.wait()
            return carry

        lax.fori_loop(0, nsuper, body, 0)
        plsc.subcore_barrier()
        pltpu.sync_copy(deg_sp.at[pl.ds(r0, rpt)], out_hbm.at[c].at[pl.ds(r0, rpt)])

    return deg_kernel


def _make_scatter_kernel(n_pad, e, d):
    rpt = n_pad // _NS
    ept = e // _NW
    sb = 40
    sup = sb * _K
    nsuper = ept // sup

    @functools.partial(
        pl.kernel,
        out_type=jax.ShapeDtypeStruct((_NC, n_pad, d), jnp.float32),
        mesh=_sc_mesh(),
        scratch_types=(
            [pltpu.VMEM((ept,), jnp.int32)]
            + [pltpu.VMEM((sb,), jnp.int32) for _ in range(_K)]
            + [
                pltpu.VMEM((sup, d), jnp.float32),
                pltpu.VMEM_SHARED((n_pad, d), jnp.float32),
                pltpu.SemaphoreType.DMA,
                pltpu.SemaphoreType.DMA,
            ]
        ),
    )
    def scatter_kernel(src_hbm, dst_hbm, g_hbm, zeros_hbm, out_hbm, *scr):
        src_big = scr[0]
        didx = scr[1:1 + _K]
        rows_v, acc_sp, gsem, ssem = scr[1 + _K:]
        c = lax.axis_index("c")
        s = lax.axis_index("s")
        wid = c * _NS + s
        r0 = s * rpt
        e0 = wid * ept
        pltpu.sync_copy(zeros_hbm, acc_sp.at[pl.ds(r0, rpt)])
        pltpu.sync_copy(src_hbm.at[pl.ds(e0, ept)], src_big)
        plsc.subcore_barrier()

        def body(m, carry):
            base = m * sup
            fired = [
                pltpu.async_copy(dst_hbm.at[pl.ds(e0 + base + i * sb, sb)],
                                 didx[i], gsem)
                for i in range(_K)
            ] + [
                pltpu.async_copy(g_hbm.at[src_big.at[pl.ds(base + i * sb, sb)]],
                                 rows_v.at[pl.ds(i * sb, sb)], gsem)
                for i in range(_K)
            ]
            for f in fired:
                f.wait()
            sc = [
                pltpu.async_copy(rows_v.at[pl.ds(i * sb, sb)],
                                 acc_sp.at[didx[i]], ssem, add=True)
                for i in range(_K)
            ]
            for f in sc:
                f.wait()
            return carry

        lax.fori_loop(0, nsuper, body, 0)
        plsc.subcore_barrier()
        pltpu.sync_copy(acc_sp.at[pl.ds(r0, rpt)], out_hbm.at[c].at[pl.ds(r0, rpt)])

    return scatter_kernel


def _tc_scale_body(x_ref, w_ref, dp_ref, g_ref):
    h = jnp.dot(x_ref[...], w_ref[...], preferred_element_type=jnp.float32)
    deg = dp_ref[0, :, 0:1] + dp_ref[1, :, 0:1] + 1.0
    g_ref[...] = h * lax.rsqrt(deg)


def _tc_final_body(s_ref, g_ref, dp_ref, b_ref, o_ref):
    deg = dp_ref[0, :, 0:1] + dp_ref[1, :, 0:1] + 1.0
    dinv = lax.rsqrt(deg)
    agg = s_ref[0] + s_ref[1] + g_ref[...]
    o_ref[...] = jnp.maximum(dinv * agg + b_ref[...], 0.0)


def kernel(x, edge_index, W, b):
    n, d = x.shape
    e = edge_index.shape[1]
    src = edge_index[0]
    dst = edge_index[1]
    n_pad = _pad_rows(n)
    rpt = n_pad // _NS

    zeros16 = jnp.zeros((rpt, _DEGW), jnp.float32)
    ones16 = jnp.ones((_B, _DEGW), jnp.float32)
    zeros_d = jnp.zeros((rpt, d), jnp.float32)

    deg_partials = _make_deg_kernel(n_pad, e)(dst, zeros16, ones16)

    blk = 2000
    grid = n // blk
    g = pl.pallas_call(
        _tc_scale_body,
        out_shape=jax.ShapeDtypeStruct((n, d), jnp.float32),
        grid=(grid,),
        in_specs=[
            pl.BlockSpec((blk, d), lambda i: (i, 0)),
            pl.BlockSpec((d, d), lambda i: (0, 0)),
            pl.BlockSpec((_NC, blk, _DEGW), lambda i: (0, i, 0)),
        ],
        out_specs=pl.BlockSpec((blk, d), lambda i: (i, 0)),
    )(x, W, deg_partials)

    s_partials = _make_scatter_kernel(n_pad, e, d)(src, dst, g, zeros_d)

    out = pl.pallas_call(
        _tc_final_body,
        out_shape=jax.ShapeDtypeStruct((n, d), jnp.float32),
        grid=(grid,),
        in_specs=[
            pl.BlockSpec((_NC, blk, d), lambda i: (0, i, 0)),
            pl.BlockSpec((blk, d), lambda i: (i, 0)),
            pl.BlockSpec((_NC, blk, _DEGW), lambda i: (0, i, 0)),
            pl.BlockSpec((1, d), lambda i: (0, 0)),
        ],
        out_specs=pl.BlockSpec((blk, d), lambda i: (i, 0)),
    )(s_partials, g, deg_partials, b.reshape(1, d))

    return out


# speedup vs baseline: 29.6464x; 1.1004x over previous
import functools

import jax
import jax.numpy as jnp
from jax import lax
from jax.experimental import pallas as pl
from jax.experimental.pallas import tpu as pltpu
from jax.experimental.pallas import tpu_sc as plsc

_NC = 2
_NS = 16
_NW = _NC * _NS
_B = 80
_DEGW = 128


def _sc_mesh():
    return plsc.VectorSubcoreMesh(core_axis_name="c", subcore_axis_name="s")


def _pad_rows(n):
    step = 8 * _NS
    return ((n + step - 1) // step) * step


_K = 5


def _make_deg_kernel(n_pad, e):
    rpt = n_pad // _NS
    ept = e // _NW
    nsuper = ept // (_B * _K)

    @functools.partial(
        pl.kernel,
        out_type=jax.ShapeDtypeStruct((_NC, n_pad, _DEGW), jnp.float32),
        mesh=_sc_mesh(),
        scratch_types=(
            [pltpu.VMEM((_B,), jnp.int32) for _ in range(_K)]
            + [
                pltpu.VMEM((_B, _DEGW), jnp.float32),
                pltpu.VMEM_SHARED((n_pad, _DEGW), jnp.float32),
                pltpu.SemaphoreType.DMA,
                pltpu.SemaphoreType.DMA,
            ]
        ),
    )
    def deg_kernel(dst_hbm, zeros_hbm, ones_hbm, out_hbm, *scr):
        didx = scr[:_K]
        ones_v, deg_sp, isem, ssem = scr[_K:]
        c = lax.axis_index("c")
        s = lax.axis_index("s")
        wid = c * _NS + s
        r0 = s * rpt
        pltpu.sync_copy(zeros_hbm, deg_sp.at[pl.ds(r0, rpt)])
        pltpu.sync_copy(ones_hbm, ones_v)
        plsc.subcore_barrier()
        e0 = wid * ept

        def body(m, carry):
            base = e0 + m * (_B * _K)
            ic = [
                pltpu.async_copy(dst_hbm.at[pl.ds(base + i * _B, _B)], didx[i], isem)
                for i in range(_K)
            ]
            for d in ic:
                d.wait()
            sc = [
                pltpu.async_copy(ones_v, deg_sp.at[didx[i]], ssem, add=True)
                for i in range(_K)
            ]
            for d in sc:
                d.wait()
            return carry

        lax.fori_loop(0, nsuper, body, 0)
        plsc.subcore_barrier()
        pltpu.sync_copy(deg_sp.at[pl.ds(r0, rpt)], out_hbm.at[c].at[pl.ds(r0, rpt)])

    return deg_kernel


def _make_scatter_kernel(n_pad, e, d):
    rpt = n_pad // _NS
    ept = e // _NW
    sb = 40
    sup = sb * _K
    nsuper = ept // sup

    ka, kb = 3, 2
    ngroup = ept // (sb * _K)
    assert ka + kb == _K

    @functools.partial(
        pl.kernel,
        out_type=jax.ShapeDtypeStruct((_NC, n_pad, d), jnp.float32),
        mesh=_sc_mesh(),
        scratch_types=(
            [pltpu.VMEM((ept,), jnp.int32)]
            + [pltpu.VMEM((sb,), jnp.int32) for _ in range(_K)]
            + [
                pltpu.VMEM((ka * sb, d), jnp.float32),
                pltpu.VMEM((kb * sb, d), jnp.float32),
                pltpu.VMEM_SHARED((n_pad, d), jnp.float32),
                pltpu.SemaphoreType.DMA,
                pltpu.SemaphoreType.DMA,
                pltpu.SemaphoreType.DMA,
                pltpu.SemaphoreType.DMA,
            ]
        ),
    )
    def scatter_kernel(src_hbm, dst_hbm, g_hbm, zeros_hbm, out_hbm, *scr):
        src_big = scr[0]
        didx_a = scr[1:1 + ka]
        didx_b = scr[1 + ka:1 + _K]
        rows_a, rows_b, acc_sp, gsem_a, gsem_b, ssem_a, ssem_b = scr[1 + _K:]
        c = lax.axis_index("c")
        s = lax.axis_index("s")
        wid = c * _NS + s
        r0 = s * rpt
        e0 = wid * ept
        pltpu.sync_copy(zeros_hbm, acc_sp.at[pl.ds(r0, rpt)])
        pltpu.sync_copy(src_hbm.at[pl.ds(e0, ept)], src_big)
        plsc.subcore_barrier()

        def fire_gathers(chunk0, didx, rows, gsem, k):
            base = chunk0 * sb
            for i in range(k):
                pltpu.async_copy(dst_hbm.at[pl.ds(e0 + base + i * sb, sb)],
                                 didx[i], gsem)
                pltpu.async_copy(g_hbm.at[src_big.at[pl.ds(base + i * sb, sb)]],
                                 rows.at[pl.ds(i * sb, sb)], gsem)

        def drain_gathers(didx, rows, gsem, k):
            for i in range(k):
                pltpu.make_async_copy(dst_hbm.at[pl.ds(0, sb)],
                                      didx[i], gsem).wait()
                pltpu.make_async_copy(g_hbm.at[pl.ds(0, sb)],
                                      rows.at[pl.ds(i * sb, sb)], gsem).wait()

        def fire_scatters(didx, rows, ssem, k):
            for i in range(k):
                pltpu.async_copy(rows.at[pl.ds(i * sb, sb)],
                                 acc_sp.at[didx[i]], ssem, add=True)

        def drain_scatters(rows, ssem, k):
            for i in range(k):
                pltpu.make_async_copy(g_hbm.at[pl.ds(0, sb)],
                                      rows.at[pl.ds(i * sb, sb)], ssem).wait()

        fire_gathers(0, didx_a, rows_a, gsem_a, ka)

        def body(g, carry):
            cbase = g * _K

            @pl.when(g > 0)
            def _():
                drain_scatters(rows_b, ssem_b, kb)

            fire_gathers(cbase + ka, didx_b, rows_b, gsem_b, kb)
            drain_gathers(didx_a, rows_a, gsem_a, ka)
            fire_scatters(didx_a, rows_a, ssem_a, ka)
            drain_scatters(rows_a, ssem_a, ka)

            @pl.when(g < ngroup - 1)
            def _():
                fire_gathers(cbase + _K, didx_a, rows_a, gsem_a, ka)

            drain_gathers(didx_b, rows_b, gsem_b, kb)
            fire_scatters(didx_b, rows_b, ssem_b, kb)
            return carry

        lax.fori_loop(0, ngroup, body, 0)
        drain_scatters(rows_b, ssem_b, kb)
        plsc.subcore_barrier()
        pltpu.sync_copy(acc_sp.at[pl.ds(r0, rpt)], out_hbm.at[c].at[pl.ds(r0, rpt)])

    return scatter_kernel


def _tc_scale_body(x_ref, w_ref, dp_ref, g_ref):
    h = jnp.dot(x_ref[...], w_ref[...], preferred_element_type=jnp.float32)
    deg = dp_ref[0, :, 0:1] + dp_ref[1, :, 0:1] + 1.0
    g_ref[...] = h * lax.rsqrt(deg)


def _tc_final_body(s_ref, g_ref, dp_ref, b_ref, o_ref):
    deg = dp_ref[0, :, 0:1] + dp_ref[1, :, 0:1] + 1.0
    dinv = lax.rsqrt(deg)
    agg = s_ref[0] + s_ref[1] + g_ref[...]
    o_ref[...] = jnp.maximum(dinv * agg + b_ref[...], 0.0)


def kernel(x, edge_index, W, b):
    n, d = x.shape
    e = edge_index.shape[1]
    src = edge_index[0]
    dst = edge_index[1]
    n_pad = _pad_rows(n)
    rpt = n_pad // _NS

    zeros16 = jnp.zeros((rpt, _DEGW), jnp.float32)
    ones16 = jnp.ones((_B, _DEGW), jnp.float32)
    zeros_d = jnp.zeros((rpt, d), jnp.float32)

    deg_partials = _make_deg_kernel(n_pad, e)(dst, zeros16, ones16)

    blk = 2000
    grid = n // blk
    g = pl.pallas_call(
        _tc_scale_body,
        out_shape=jax.ShapeDtypeStruct((n, d), jnp.float32),
        grid=(grid,),
        in_specs=[
            pl.BlockSpec((blk, d), lambda i: (i, 0)),
            pl.BlockSpec((d, d), lambda i: (0, 0)),
            pl.BlockSpec((_NC, blk, _DEGW), lambda i: (0, i, 0)),
        ],
        out_specs=pl.BlockSpec((blk, d), lambda i: (i, 0)),
    )(x, W, deg_partials)

    s_partials = _make_scatter_kernel(n_pad, e, d)(src, dst, g, zeros_d)

    out = pl.pallas_call(
        _tc_final_body,
        out_shape=jax.ShapeDtypeStruct((n, d), jnp.float32),
        grid=(grid,),
        in_specs=[
            pl.BlockSpec((_NC, blk, d), lambda i: (0, i, 0)),
            pl.BlockSpec((blk, d), lambda i: (i, 0)),
            pl.BlockSpec((_NC, blk, _DEGW), lambda i: (0, i, 0)),
            pl.BlockSpec((1, d), lambda i: (0, 0)),
        ],
        out_specs=pl.BlockSpec((blk, d), lambda i: (i, 0)),
    )(s_partials, g, deg_partials, b.reshape(1, d))

    return out


# speedup vs baseline: 35.8341x; 1.2087x over previous
import functools

import jax
import jax.numpy as jnp
from jax import lax
from jax.experimental import pallas as pl
from jax.experimental.pallas import tpu as pltpu
from jax.experimental.pallas import tpu_sc as plsc

_NC = 2
_NS = 16
_NW = _NC * _NS
_B = 80
_DEGW = 16


def _sc_mesh():
    return plsc.VectorSubcoreMesh(core_axis_name="c", subcore_axis_name="s")


def _pad_rows(n, align=8):
    step = align * _NS
    return ((n + step - 1) // step) * step


_K = 5


def _make_deg_kernel(n_pad, e):
    rpt = n_pad // _NS
    ept = e // _NW
    nsuper = ept // (_B * _K)

    @functools.partial(
        pl.kernel,
        out_type=jax.ShapeDtypeStruct((_NC, n_pad, _DEGW), jnp.float32),
        mesh=_sc_mesh(),
        scratch_types=(
            [pltpu.VMEM((_B,), jnp.int32) for _ in range(_K)]
            + [
                pltpu.VMEM((_B, _DEGW), jnp.float32),
                pltpu.VMEM_SHARED((n_pad, _DEGW), jnp.float32),
                pltpu.SemaphoreType.DMA,
                pltpu.SemaphoreType.DMA,
            ]
        ),
        compiler_params=pltpu.CompilerParams(use_tc_tiling_on_sc=False),
    )
    def deg_kernel(dst_hbm, zeros_hbm, ones_hbm, out_hbm, *scr):
        didx = scr[:_K]
        ones_v, deg_sp, isem, ssem = scr[_K:]
        c = lax.axis_index("c")
        s = lax.axis_index("s")
        wid = c * _NS + s
        r0 = s * rpt
        pltpu.sync_copy(zeros_hbm, deg_sp.at[pl.ds(r0, rpt)])
        pltpu.sync_copy(ones_hbm, ones_v)
        plsc.subcore_barrier()
        e0 = wid * ept

        def body(m, carry):
            base = e0 + m * (_B * _K)
            ic = [
                pltpu.async_copy(dst_hbm.at[pl.ds(base + i * _B, _B)], didx[i], isem)
                for i in range(_K)
            ]
            for d in ic:
                d.wait()
            sc = [
                pltpu.async_copy(ones_v, deg_sp.at[didx[i]], ssem, add=True)
                for i in range(_K)
            ]
            for d in sc:
                d.wait()
            return carry

        lax.fori_loop(0, nsuper, body, 0)
        plsc.subcore_barrier()
        pltpu.sync_copy(deg_sp.at[pl.ds(r0, rpt)], out_hbm.at[c].at[pl.ds(r0, rpt)])

    return deg_kernel


def _make_scatter_kernel(n_pad, e, d):
    rpt = n_pad // _NS
    ept = e // _NW
    sb = 40
    sup = sb * _K
    nsuper = ept // sup

    ka, kb = 3, 2
    ngroup = ept // (sb * _K)
    assert ka + kb == _K

    @functools.partial(
        pl.kernel,
        out_type=jax.ShapeDtypeStruct((_NC, n_pad, d), jnp.float32),
        mesh=_sc_mesh(),
        scratch_types=(
            [pltpu.VMEM((ept,), jnp.int32)]
            + [pltpu.VMEM((sb,), jnp.int32) for _ in range(_K)]
            + [
                pltpu.VMEM((ka * sb, d), jnp.float32),
                pltpu.VMEM((kb * sb, d), jnp.float32),
                pltpu.VMEM_SHARED((n_pad, d), jnp.float32),
                pltpu.SemaphoreType.DMA,
                pltpu.SemaphoreType.DMA,
                pltpu.SemaphoreType.DMA,
                pltpu.SemaphoreType.DMA,
            ]
        ),
    )
    def scatter_kernel(src_hbm, dst_hbm, g_hbm, zeros_hbm, out_hbm, *scr):
        src_big = scr[0]
        didx_a = scr[1:1 + ka]
        didx_b = scr[1 + ka:1 + _K]
        rows_a, rows_b, acc_sp, gsem_a, gsem_b, ssem_a, ssem_b = scr[1 + _K:]
        c = lax.axis_index("c")
        s = lax.axis_index("s")
        wid = c * _NS + s
        r0 = s * rpt
        e0 = wid * ept
        pltpu.sync_copy(zeros_hbm, acc_sp.at[pl.ds(r0, rpt)])
        pltpu.sync_copy(src_hbm.at[pl.ds(e0, ept)], src_big)
        plsc.subcore_barrier()

        def fire_gathers(chunk0, didx, rows, gsem, k):
            base = chunk0 * sb
            for i in range(k):
                pltpu.async_copy(dst_hbm.at[pl.ds(e0 + base + i * sb, sb)],
                                 didx[i], gsem)
                pltpu.async_copy(g_hbm.at[src_big.at[pl.ds(base + i * sb, sb)]],
                                 rows.at[pl.ds(i * sb, sb)], gsem)

        def drain_gathers(didx, rows, gsem, k):
            for i in range(k):
                pltpu.make_async_copy(dst_hbm.at[pl.ds(0, sb)],
                                      didx[i], gsem).wait()
                pltpu.make_async_copy(g_hbm.at[pl.ds(0, sb)],
                                      rows.at[pl.ds(i * sb, sb)], gsem).wait()

        def fire_scatters(didx, rows, ssem, k):
            for i in range(k):
                pltpu.async_copy(rows.at[pl.ds(i * sb, sb)],
                                 acc_sp.at[didx[i]], ssem, add=True)

        def drain_scatters(rows, ssem, k):
            for i in range(k):
                pltpu.make_async_copy(g_hbm.at[pl.ds(0, sb)],
                                      rows.at[pl.ds(i * sb, sb)], ssem).wait()

        fire_gathers(0, didx_a, rows_a, gsem_a, ka)

        def body(g, carry):
            cbase = g * _K

            @pl.when(g > 0)
            def _():
                drain_scatters(rows_b, ssem_b, kb)

            fire_gathers(cbase + ka, didx_b, rows_b, gsem_b, kb)
            drain_gathers(didx_a, rows_a, gsem_a, ka)
            fire_scatters(didx_a, rows_a, ssem_a, ka)
            drain_scatters(rows_a, ssem_a, ka)

            @pl.when(g < ngroup - 1)
            def _():
                fire_gathers(cbase + _K, didx_a, rows_a, gsem_a, ka)

            drain_gathers(didx_b, rows_b, gsem_b, kb)
            fire_scatters(didx_b, rows_b, ssem_b, kb)
            return carry

        lax.fori_loop(0, ngroup, body, 0)
        drain_scatters(rows_b, ssem_b, kb)
        plsc.subcore_barrier()
        pltpu.sync_copy(acc_sp.at[pl.ds(r0, rpt)], out_hbm.at[c].at[pl.ds(r0, rpt)])

    return scatter_kernel


def _deg_from_partials(dp_ref):
    return dp_ref[0, :, 0:1] + dp_ref[1, :, 0:1] + 1.0


def _tc_scale_body(x_ref, w_ref, dp_ref, g_ref):
    h = jnp.dot(x_ref[...], w_ref[...], preferred_element_type=jnp.float32)
    g_ref[...] = h * lax.rsqrt(_deg_from_partials(dp_ref))


def _tc_final_body(s_ref, g_ref, dp_ref, b_ref, o_ref):
    dinv = lax.rsqrt(_deg_from_partials(dp_ref))
    agg = s_ref[0] + s_ref[1] + g_ref[...]
    o_ref[...] = jnp.maximum(dinv * agg + b_ref[...], 0.0)


def kernel(x, edge_index, W, b):
    n, d = x.shape
    e = edge_index.shape[1]
    src = edge_index[0]
    dst = edge_index[1]
    n_pad = _pad_rows(n)
    rpt = n_pad // _NS
    zeros16 = jnp.zeros((rpt, _DEGW), jnp.float32)
    ones16 = jnp.ones((_B, _DEGW), jnp.float32)
    zeros_d = jnp.zeros((rpt, d), jnp.float32)

    deg_partials = _make_deg_kernel(n_pad, e)(dst, zeros16, ones16)

    blk = 2000
    grid = n // blk
    g = pl.pallas_call(
        _tc_scale_body,
        out_shape=jax.ShapeDtypeStruct((n, d), jnp.float32),
        grid=(grid,),
        in_specs=[
            pl.BlockSpec((blk, d), lambda i: (i, 0)),
            pl.BlockSpec((d, d), lambda i: (0, 0)),
            pl.BlockSpec((_NC, blk, _DEGW), lambda i: (0, i, 0)),
        ],
        out_specs=pl.BlockSpec((blk, d), lambda i: (i, 0)),
    )(x, W, deg_partials)

    s_partials = _make_scatter_kernel(n_pad, e, d)(src, dst, g, zeros_d)

    out = pl.pallas_call(
        _tc_final_body,
        out_shape=jax.ShapeDtypeStruct((n, d), jnp.float32),
        grid=(grid,),
        in_specs=[
            pl.BlockSpec((_NC, blk, d), lambda i: (0, i, 0)),
            pl.BlockSpec((blk, d), lambda i: (i, 0)),
            pl.BlockSpec((_NC, blk, _DEGW), lambda i: (0, i, 0)),
            pl.BlockSpec((1, d), lambda i: (0, 0)),
        ],
        out_specs=pl.BlockSpec((blk, d), lambda i: (i, 0)),
    )(s_partials, g, deg_partials, b.reshape(1, d))

    return out


# speedup vs baseline: 36.2791x; 1.0124x over previous
import functools

import jax
import jax.numpy as jnp
from jax import lax
from jax.experimental import pallas as pl
from jax.experimental.pallas import tpu as pltpu
from jax.experimental.pallas import tpu_sc as plsc

_NC = 2
_NS = 16
_NW = _NC * _NS
_B = 80
_DEGW = 16


def _sc_mesh():
    return plsc.VectorSubcoreMesh(core_axis_name="c", subcore_axis_name="s")


def _pad_rows(n, align=8):
    step = align * _NS
    return ((n + step - 1) // step) * step


_K = 5


def _make_deg_kernel(n_pad, e):
    rpt = n_pad // _NS
    ept = e // _NW
    nsuper = ept // (_B * _K)
    ka, kb = 3, 2
    assert ka + kb == _K

    @functools.partial(
        pl.kernel,
        out_type=jax.ShapeDtypeStruct((_NC, n_pad, _DEGW), jnp.float32),
        mesh=_sc_mesh(),
        scratch_types=(
            [pltpu.VMEM((_B,), jnp.int32) for _ in range(_K)]
            + [
                pltpu.VMEM((_B, _DEGW), jnp.float32),
                pltpu.VMEM_SHARED((n_pad, _DEGW), jnp.float32),
                pltpu.SemaphoreType.DMA,
                pltpu.SemaphoreType.DMA,
                pltpu.SemaphoreType.DMA,
                pltpu.SemaphoreType.DMA,
            ]
        ),
        compiler_params=pltpu.CompilerParams(use_tc_tiling_on_sc=False),
    )
    def deg_kernel(dst_hbm, zeros_hbm, ones_hbm, out_hbm, *scr):
        didx = scr[:_K]
        didx_a, didx_b = didx[:ka], didx[ka:]
        ones_v, deg_sp, isem_a, isem_b, ssem_a, ssem_b = scr[_K:]
        c = lax.axis_index("c")
        s = lax.axis_index("s")
        wid = c * _NS + s
        r0 = s * rpt
        pltpu.sync_copy(zeros_hbm, deg_sp.at[pl.ds(r0, rpt)])
        pltpu.sync_copy(ones_hbm, ones_v)
        plsc.subcore_barrier()
        e0 = wid * ept

        def fire_idx(chunk0, didxs, isem):
            for i, dref in enumerate(didxs):
                pltpu.async_copy(
                    dst_hbm.at[pl.ds(e0 + (chunk0 + i) * _B, _B)], dref, isem)

        def drain_idx(didxs, isem):
            for dref in didxs:
                pltpu.make_async_copy(dst_hbm.at[pl.ds(0, _B)], dref, isem).wait()

        def fire_scatters(didxs, ssem):
            for dref in didxs:
                pltpu.async_copy(ones_v, deg_sp.at[dref], ssem, add=True)

        def drain_scatters(didxs, ssem):
            for _ in didxs:
                pltpu.make_async_copy(ones_hbm, ones_v, ssem).wait()

        fire_idx(0, didx_a, isem_a)

        def body(g, carry):
            cbase = g * _K

            @pl.when(g > 0)
            def _():
                drain_scatters(didx_b, ssem_b)

            fire_idx(cbase + ka, didx_b, isem_b)
            drain_idx(didx_a, isem_a)
            fire_scatters(didx_a, ssem_a)
            drain_scatters(didx_a, ssem_a)

            @pl.when(g < nsuper - 1)
            def _():
                fire_idx(cbase + _K, didx_a, isem_a)

            drain_idx(didx_b, isem_b)
            fire_scatters(didx_b, ssem_b)
            return carry

        lax.fori_loop(0, nsuper, body, 0)
        drain_scatters(didx_b, ssem_b)
        plsc.subcore_barrier()
        pltpu.sync_copy(deg_sp.at[pl.ds(r0, rpt)], out_hbm.at[c].at[pl.ds(r0, rpt)])

    return deg_kernel


def _make_scatter_kernel(n_pad, e, d):
    rpt = n_pad // _NS
    ept = e // _NW
    sb = 40
    sup = sb * _K
    nsuper = ept // sup

    ka, kb = 3, 2
    ngroup = ept // (sb * _K)
    assert ka + kb == _K

    @functools.partial(
        pl.kernel,
        out_type=jax.ShapeDtypeStruct((_NC, n_pad, d), jnp.float32),
        mesh=_sc_mesh(),
        scratch_types=(
            [pltpu.VMEM((ept,), jnp.int32)]
            + [pltpu.VMEM((sb,), jnp.int32) for _ in range(_K)]
            + [
                pltpu.VMEM((ka * sb, d), jnp.float32),
                pltpu.VMEM((kb * sb, d), jnp.float32),
                pltpu.VMEM_SHARED((n_pad, d), jnp.float32),
                pltpu.SemaphoreType.DMA,
                pltpu.SemaphoreType.DMA,
                pltpu.SemaphoreType.DMA,
                pltpu.SemaphoreType.DMA,
            ]
        ),
    )
    def scatter_kernel(src_hbm, dst_hbm, g_hbm, zeros_hbm, out_hbm, *scr):
        src_big = scr[0]
        didx_a = scr[1:1 + ka]
        didx_b = scr[1 + ka:1 + _K]
        rows_a, rows_b, acc_sp, gsem_a, gsem_b, ssem_a, ssem_b = scr[1 + _K:]
        c = lax.axis_index("c")
        s = lax.axis_index("s")
        wid = c * _NS + s
        r0 = s * rpt
        e0 = wid * ept
        pltpu.sync_copy(zeros_hbm, acc_sp.at[pl.ds(r0, rpt)])
        pltpu.sync_copy(src_hbm.at[pl.ds(e0, ept)], src_big)
        plsc.subcore_barrier()

        def fire_gathers(chunk0, didx, rows, gsem, k):
            base = chunk0 * sb
            for i in range(k):
                pltpu.async_copy(dst_hbm.at[pl.ds(e0 + base + i * sb, sb)],
                                 didx[i], gsem)
                pltpu.async_copy(g_hbm.at[src_big.at[pl.ds(base + i * sb, sb)]],
                                 rows.at[pl.ds(i * sb, sb)], gsem)

        def drain_gathers(didx, rows, gsem, k):
            pltpu.make_async_copy(g_hbm.at[pl.ds(0, k * sb)], rows, gsem).wait()
            for i in range(k):
                pltpu.make_async_copy(dst_hbm.at[pl.ds(0, sb)],
                                      didx[i], gsem).wait()

        def fire_scatters(didx, rows, ssem, k):
            for i in range(k):
                pltpu.async_copy(rows.at[pl.ds(i * sb, sb)],
                                 acc_sp.at[didx[i]], ssem, add=True)

        def drain_scatters(rows, ssem, k):
            pltpu.make_async_copy(g_hbm.at[pl.ds(0, k * sb)], rows, ssem).wait()

        fire_gathers(0, didx_a, rows_a, gsem_a, ka)

        def body(g, carry):
            cbase = g * _K

            @pl.when(g > 0)
            def _():
                drain_scatters(rows_b, ssem_b, kb)

            fire_gathers(cbase + ka, didx_b, rows_b, gsem_b, kb)
            drain_gathers(didx_a, rows_a, gsem_a, ka)
            fire_scatters(didx_a, rows_a, ssem_a, ka)
            drain_scatters(rows_a, ssem_a, ka)

            @pl.when(g < ngroup - 1)
            def _():
                fire_gathers(cbase + _K, didx_a, rows_a, gsem_a, ka)

            drain_gathers(didx_b, rows_b, gsem_b, kb)
            fire_scatters(didx_b, rows_b, ssem_b, kb)
            return carry

        lax.fori_loop(0, ngroup, body, 0)
        drain_scatters(rows_b, ssem_b, kb)
        plsc.subcore_barrier()
        pltpu.sync_copy(acc_sp.at[pl.ds(r0, rpt)], out_hbm.at[c].at[pl.ds(r0, rpt)])

    return scatter_kernel


def _deg_from_partials(dp_ref):
    return dp_ref[0, :, 0:1] + dp_ref[1, :, 0:1] + 1.0


def _tc_scale_body(x_ref, w_ref, dp_ref, g_ref):
    h = jnp.dot(x_ref[...], w_ref[...], preferred_element_type=jnp.float32)
    g_ref[...] = h * lax.rsqrt(_deg_from_partials(dp_ref))


def _tc_final_body(s_ref, g_ref, dp_ref, b_ref, o_ref):
    dinv = lax.rsqrt(_deg_from_partials(dp_ref))
    agg = s_ref[0] + s_ref[1] + g_ref[...]
    o_ref[...] = jnp.maximum(dinv * agg + b_ref[...], 0.0)


def kernel(x, edge_index, W, b):
    n, d = x.shape
    e = edge_index.shape[1]
    src = edge_index[0]
    dst = edge_index[1]
    n_pad = _pad_rows(n)
    rpt = n_pad // _NS
    zeros16 = jnp.zeros((rpt, _DEGW), jnp.float32)
    ones16 = jnp.ones((_B, _DEGW), jnp.float32)
    zeros_d = jnp.zeros((rpt, d), jnp.float32)

    deg_partials = _make_deg_kernel(n_pad, e)(dst, zeros16, ones16)

    blk = 2000
    grid = n // blk
    g = pl.pallas_call(
        _tc_scale_body,
        out_shape=jax.ShapeDtypeStruct((n, d), jnp.float32),
        grid=(grid,),
        in_specs=[
            pl.BlockSpec((blk, d), lambda i: (i, 0)),
            pl.BlockSpec((d, d), lambda i: (0, 0)),
            pl.BlockSpec((_NC, blk, _DEGW), lambda i: (0, i, 0)),
        ],
        out_specs=pl.BlockSpec((blk, d), lambda i: (i, 0)),
    )(x, W, deg_partials)

    s_partials = _make_scatter_kernel(n_pad, e, d)(src, dst, g, zeros_d)

    out = pl.pallas_call(
        _tc_final_body,
        out_shape=jax.ShapeDtypeStruct((n, d), jnp.float32),
        grid=(grid,),
        in_specs=[
            pl.BlockSpec((_NC, blk, d), lambda i: (0, i, 0)),
            pl.BlockSpec((blk, d), lambda i: (i, 0)),
            pl.BlockSpec((_NC, blk, _DEGW), lambda i: (0, i, 0)),
            pl.BlockSpec((1, d), lambda i: (0, 0)),
        ],
        out_specs=pl.BlockSpec((blk, d), lambda i: (i, 0)),
    )(s_partials, g, deg_partials, b.reshape(1, d))

    return out
